# Initial kernel scaffold; baseline (speedup 1.0000x reference)
import jax
import jax.numpy as jnp
from jax.experimental import pallas as pl


def kernel(g_r1, g_r2, g_r1_mask, g_r2_mask):
    raise NotImplementedError("write your pallas kernel here")


# speedup vs baseline: 1.5631x; 1.5631x over previous
import functools

import jax
import jax.numpy as jnp
from jax import lax
from jax.experimental import pallas as pl
from jax.experimental.pallas import tpu as pltpu
from jax.experimental.pallas import tpu_sc as plsc

B_, L1_, L2_, D_ = 16, 512, 2048, 768
K_ = 256
_NEG = -10000000000.0


def _tc_body(g1_ref, g2_ref, mrow_ref, mcol_ref, gidx_ref, xmask_ref):
    b = pl.program_id(0)
    g1 = g1_ref[0]
    g2 = g2_ref[0]
    mrow = mrow_ref[0]
    mcol = mcol_ref[0]

    e = lax.dot_general(g1, g2, (((1,), (1,)), ((), ())),
                        preferred_element_type=jnp.float32)
    e = jnp.where(mrow > 0.5, e, _NEG)
    m = jnp.max(e, axis=1, keepdims=True)
    p = jnp.exp(e - m)
    s = jnp.sum(p, axis=1, keepdims=True)
    a = p / s
    w_row = jnp.sum(a, axis=0, keepdims=True)

    w8 = jnp.broadcast_to(w_row, (8, L2_))
    eye8 = (lax.broadcasted_iota(jnp.int32, (8, 8), 0)
            == lax.broadcasted_iota(jnp.int32, (8, 8), 1)).astype(jnp.float32)
    wT = lax.dot_general(w8, eye8, (((0,), (0,)), ((), ())),
                         precision=lax.Precision.HIGHEST,
                         preferred_element_type=jnp.float32)
    w_col = wT[:, 0:1]

    CH = 512
    idx_sum = jnp.zeros((1, K_), jnp.int32)
    msk_sum = jnp.zeros((1, K_), jnp.float32)
    for c in range(L2_ // CH):
        wj = w_col[c * CH:(c + 1) * CH, :]
        k_row = lax.broadcasted_iota(jnp.int32, (CH, L2_), 1)
        j_col = c * CH + lax.broadcasted_iota(jnp.int32, (CH, L2_), 0)
        gt = (w_row > wj)
        eq = (w_row == wj) & (k_row < j_col)
        cnt = jnp.sum(gt.astype(jnp.int32) + eq.astype(jnp.int32),
                      axis=1, keepdims=True)
        r_iota = lax.broadcasted_iota(jnp.int32, (CH, K_), 1)
        oh = (cnt == r_iota)
        jg = c * CH + lax.broadcasted_iota(jnp.int32, (CH, K_), 0)
        idx_sum = idx_sum + jnp.sum(jnp.where(oh, jg, 0),
                                    axis=0, keepdims=True)
        mj = mcol[c * CH:(c + 1) * CH, :]
        msk_sum = msk_sum + jnp.sum(jnp.where(oh, mj, 0.0),
                                    axis=0, keepdims=True)

    gidx_ref[...] = (idx_sum + b * L2_).reshape(1, 1, K_)
    xmask_ref[...] = msk_sum.reshape(1, 1, K_)


def _tc_topk(g_r1, g_r2, mask_row, mask_col, interpret=False):
    return pl.pallas_call(
        _tc_body,
        grid=(B_,),
        in_specs=[
            pl.BlockSpec((1, L1_, D_), lambda b: (b, 0, 0)),
            pl.BlockSpec((1, L2_, D_), lambda b: (b, 0, 0)),
            pl.BlockSpec((1, 1, L2_), lambda b: (b, 0, 0)),
            pl.BlockSpec((1, L2_, 1), lambda b: (b, 0, 0)),
        ],
        out_specs=[
            pl.BlockSpec((1, 1, K_), lambda b: (b, 0, 0)),
            pl.BlockSpec((1, 1, K_), lambda b: (b, 0, 0)),
        ],
        out_shape=[
            jax.ShapeDtypeStruct((B_, 1, K_), jnp.int32),
            jax.ShapeDtypeStruct((B_, 1, K_), jnp.float32),
        ],
        interpret=interpret,
    )(g_r1, g_r2, mask_row, mask_col)


_NC, _NS = 2, 16
_NW = _NC * _NS
_ROWS = B_ * K_
_BPW = _ROWS // _NW


@functools.lru_cache(maxsize=1)
def _sc_gather_fn():
    @functools.partial(
        pl.kernel,
        mesh=plsc.VectorSubcoreMesh(core_axis_name="c", subcore_axis_name="s"),
        out_type=jax.ShapeDtypeStruct((_ROWS, D_), jnp.float32),
        scratch_types=[
            pltpu.VMEM((_BPW,), jnp.int32),
            pltpu.VMEM((_BPW, D_), jnp.float32),
            pltpu.SemaphoreType.DMA,
        ],
    )
    def _sc_gather(table_hbm, idx_hbm, out_hbm, idx_v, rows_v, sem):
        wid = lax.axis_index("s") * _NC + lax.axis_index("c")
        base = wid * _BPW
        pltpu.sync_copy(idx_hbm.at[pl.ds(base, _BPW)], idx_v)
        pltpu.async_copy(table_hbm.at[idx_v], rows_v, sem).wait()
        pltpu.sync_copy(rows_v, out_hbm.at[pl.ds(base, _BPW)])

    return _sc_gather


def kernel(g_r1, g_r2, g_r1_mask, g_r2_mask):
    m2 = g_r2_mask.reshape(B_, L2_).astype(jnp.float32)
    gidx, xmask = _tc_topk(g_r1, g_r2,
                           m2.reshape(B_, 1, L2_), m2.reshape(B_, L2_, 1))
    table = g_r2.reshape(B_ * L2_, D_)
    xflat = _sc_gather_fn()(table, gidx.reshape(_ROWS))
    x = xflat.reshape(B_, K_, D_)
    x_mask = (xmask.reshape(B_, 1, 1, K_) > 0.5)
    return (x, x_mask)


# speedup vs baseline: 1.6053x; 1.0271x over previous
import functools

import jax
import jax.numpy as jnp
from jax import lax
from jax.experimental import pallas as pl
from jax.experimental.pallas import tpu as pltpu
from jax.experimental.pallas import tpu_sc as plsc

B_, L1_, L2_, D_ = 16, 512, 2048, 768
K_ = 256
_NEG = -10000000000.0


def _tc_body(g1_ref, g2_ref, mrow_ref, mcol_ref, gidx_ref, xmask_ref):
    b = pl.program_id(0)
    g1 = g1_ref[0]
    g2 = g2_ref[0]
    mrow = mrow_ref[0]
    mcol = mcol_ref[0]

    e = lax.dot_general(g1, g2, (((1,), (1,)), ((), ())),
                        preferred_element_type=jnp.float32)
    e = jnp.where(mrow > 0.5, e, _NEG)
    m = jnp.max(e, axis=1, keepdims=True)
    p = jnp.exp(e - m)
    s = jnp.sum(p, axis=1, keepdims=True)
    a = p / s
    w_row = jnp.sum(a, axis=0, keepdims=True)

    w8 = jnp.broadcast_to(w_row, (8, L2_))
    eye8 = (lax.broadcasted_iota(jnp.int32, (8, 8), 0)
            == lax.broadcasted_iota(jnp.int32, (8, 8), 1)).astype(jnp.float32)
    wT = lax.dot_general(w8, eye8, (((0,), (0,)), ((), ())),
                         precision=lax.Precision.HIGHEST,
                         preferred_element_type=jnp.float32)
    w_col = wT[:, 0:1]

    CH = 512
    k_row = lax.broadcasted_iota(jnp.int32, (CH, L2_), 1)
    r_iota = lax.broadcasted_iota(jnp.int32, (CH, K_), 1)
    jg = lax.broadcasted_iota(jnp.int32, (CH, K_), 0)
    idx_sum = jnp.zeros((1, K_), jnp.int32)
    msk_sum = jnp.zeros((1, K_), jnp.float32)
    for c in range(L2_ // CH):
        wj = w_col[c * CH:(c + 1) * CH, :]
        j_col = c * CH + lax.broadcasted_iota(jnp.int32, (CH, L2_), 0)
        before = (w_row > wj) | ((w_row == wj) & (k_row < j_col))
        cnt = jnp.sum(before.astype(jnp.int32), axis=1, keepdims=True)
        oh = (cnt == r_iota)
        idx_sum = idx_sum + jnp.sum(jnp.where(oh, c * CH + jg, 0),
                                    axis=0, keepdims=True)
        mj = mcol[c * CH:(c + 1) * CH, :]
        msk_sum = msk_sum + jnp.sum(jnp.where(oh, mj, 0.0),
                                    axis=0, keepdims=True)

    gidx_ref[...] = (idx_sum + b * L2_).reshape(1, 1, K_)
    xmask_ref[...] = msk_sum.reshape(1, 1, K_)


def _tc_topk(g_r1, g_r2, mask_row, mask_col, interpret=False):
    return pl.pallas_call(
        _tc_body,
        grid=(B_,),
        in_specs=[
            pl.BlockSpec((1, L1_, D_), lambda b: (b, 0, 0)),
            pl.BlockSpec((1, L2_, D_), lambda b: (b, 0, 0)),
            pl.BlockSpec((1, 1, L2_), lambda b: (b, 0, 0)),
            pl.BlockSpec((1, L2_, 1), lambda b: (b, 0, 0)),
        ],
        out_specs=[
            pl.BlockSpec((1, 1, K_), lambda b: (b, 0, 0)),
            pl.BlockSpec((1, 1, K_), lambda b: (b, 0, 0)),
        ],
        out_shape=[
            jax.ShapeDtypeStruct((B_, 1, K_), jnp.int32),
            jax.ShapeDtypeStruct((B_, 1, K_), jnp.float32),
        ],
        interpret=interpret,
    )(g_r1, g_r2, mask_row, mask_col)


_NC, _NS = 2, 16
_NW = _NC * _NS
_ROWS = B_ * K_
_BPW = _ROWS // _NW


@functools.lru_cache(maxsize=1)
def _sc_gather_fn():
    @functools.partial(
        pl.kernel,
        mesh=plsc.VectorSubcoreMesh(core_axis_name="c", subcore_axis_name="s"),
        out_type=jax.ShapeDtypeStruct((_ROWS, D_), jnp.float32),
        scratch_types=[
            pltpu.VMEM((_BPW,), jnp.int32),
            pltpu.VMEM((_BPW, D_), jnp.float32),
            pltpu.SemaphoreType.DMA,
        ],
    )
    def _sc_gather(table_hbm, idx_hbm, out_hbm, idx_v, rows_v, sem):
        wid = lax.axis_index("s") * _NC + lax.axis_index("c")
        base = wid * _BPW
        pltpu.sync_copy(idx_hbm.at[pl.ds(base, _BPW)], idx_v)
        pltpu.async_copy(table_hbm.at[idx_v], rows_v, sem).wait()
        pltpu.sync_copy(rows_v, out_hbm.at[pl.ds(base, _BPW)])

    return _sc_gather


def kernel(g_r1, g_r2, g_r1_mask, g_r2_mask):
    m2 = g_r2_mask.reshape(B_, L2_).astype(jnp.float32)
    gidx, xmask = _tc_topk(g_r1, g_r2,
                           m2.reshape(B_, 1, L2_), m2.reshape(B_, L2_, 1))
    table = g_r2.reshape(B_ * L2_, D_)
    xflat = _sc_gather_fn()(table, gidx.reshape(_ROWS))
    x = xflat.reshape(B_, K_, D_)
    x_mask = (xmask.reshape(B_, 1, 1, K_) > 0.5)
    return (x, x_mask)
